# Initial kernel scaffold; baseline (speedup 1.0000x reference)
import jax
import jax.numpy as jnp
from jax.experimental import pallas as pl


def kernel(x, norm_gamma, slot_norm_gamma, slot_embeds, w1, b1, w2, b2):
    raise NotImplementedError("write your pallas kernel here")


# speedup vs baseline: 1.4129x; 1.4129x over previous
import functools

import jax
import jax.numpy as jnp
from jax.experimental import pallas as pl
from jax.experimental.pallas import tpu as pltpu


def _norm_body(scale, t_ref, g_ref, o_ref):
    t = t_ref[...]
    ss = jnp.sum(t * t, axis=1, keepdims=True)
    inv = jax.lax.rsqrt(jnp.maximum(ss, 1e-24))
    o_ref[...] = (t * (inv * scale) * g_ref[...]).astype(jnp.bfloat16)


def _rmsnorm_bf16(t2, gamma, scale):
    r, d = t2.shape
    br = 512 if r % 512 == 0 else r
    return pl.pallas_call(
        functools.partial(_norm_body, scale),
        grid=(r // br,),
        in_specs=[
            pl.BlockSpec((br, d), lambda i: (i, 0)),
            pl.BlockSpec((1, d), lambda i: (0, 0)),
        ],
        out_specs=pl.BlockSpec((br, d), lambda i: (i, 0)),
        out_shape=jax.ShapeDtypeStruct((r, d), jnp.bfloat16),
    )(t2, gamma.reshape(1, d))


def _dispatch_body(nt, k, xn_ref, se_ref, slots_ref, cmax_ref, csum_ref):
    it = pl.program_id(1)
    xn = xn_ref[0]
    se = se_ref[...]
    logits = jax.lax.dot_general(
        xn, se, (((1,), (1,)), ((), ())), preferred_element_type=jnp.float32)
    tmax = jnp.max(logits, axis=0, keepdims=True)

    @pl.when(it == 0)
    def _():
        p = jnp.exp(logits - tmax)
        cmax_ref[...] = tmax
        csum_ref[...] = jnp.sum(p, axis=0, keepdims=True)
        slots_ref[0] = jax.lax.dot_general(
            p.astype(jnp.bfloat16), xn, (((0,), (0,)), ((), ())),
            preferred_element_type=jnp.float32)

    @pl.when(it != 0)
    def _():
        m_old = cmax_ref[...]
        m_new = jnp.maximum(m_old, tmax)
        p = jnp.exp(logits - m_new)
        r = jnp.exp(m_old - m_new)
        cmax_ref[...] = m_new
        csum_ref[...] = csum_ref[...] * r + jnp.sum(p, axis=0, keepdims=True)
        slots_ref[0] = slots_ref[0] * r.reshape(k, 1) + jax.lax.dot_general(
            p.astype(jnp.bfloat16), xn, (((0,), (0,)), ((), ())),
            preferred_element_type=jnp.float32)

    @pl.when(it == nt - 1)
    def _():
        slots_ref[0] = slots_ref[0] * (1.0 / csum_ref[...]).reshape(k, 1)


def _ffn_body(bsz, s, d, slots_ref, w1_ref, b1_ref, w2_ref, b2_ref, y_ref):
    ht = pl.program_id(1)
    a = slots_ref[...].astype(jnp.bfloat16).reshape(bsz * s, d)
    h = jax.lax.dot_general(
        a, w1_ref[0], (((1,), (0,)), ((), ())),
        preferred_element_type=jnp.float32) + b1_ref[0]
    g = 0.5 * h * (1.0 + jax.lax.erf(h * 0.7071067811865476))
    part = jax.lax.dot_general(
        g.astype(jnp.bfloat16), w2_ref[0], (((1,), (0,)), ((), ())),
        preferred_element_type=jnp.float32)

    @pl.when(ht == 0)
    def _():
        y_ref[...] = (part + b2_ref[0]).astype(jnp.bfloat16).reshape(bsz, s, d)

    @pl.when(ht != 0)
    def _():
        y_ref[...] = y_ref[...] + part.astype(jnp.bfloat16).reshape(bsz, s, d)


def _combine_body(xn_ref, se_ref, y_ref, out_ref):
    xn = xn_ref[0]
    se = se_ref[...]
    y = y_ref[0]
    logits = jax.lax.dot_general(
        xn, se, (((1,), (1,)), ((), ())), preferred_element_type=jnp.float32)
    rmax = jnp.max(logits, axis=1, keepdims=True)
    c = jnp.exp(logits - rmax)
    rsum = jnp.sum(c, axis=1, keepdims=True)
    c = (c * (1.0 / rsum)).astype(jnp.bfloat16)
    out_ref[0] = jax.lax.dot_general(
        c, y, (((1,), (0,)), ((), ())), preferred_element_type=jnp.float32)


def kernel(x, norm_gamma, slot_norm_gamma, slot_embeds, w1, b1, w2, b2):
    bsz, n, d = x.shape
    e, s, _ = slot_embeds.shape
    k = e * s
    dh = w1.shape[2]
    scale = float(d) ** 0.5
    bnd = 256 if n % 256 == 0 else n
    ntd = n // bnd
    bnc = 512 if n % 512 == 0 else n
    ntc = n // bnc
    dhb = dh // 2 if dh % 2 == 0 else dh
    nht = dh // dhb

    xn = _rmsnorm_bf16(x.reshape(bsz * n, d), norm_gamma, scale).reshape(bsz, n, d)
    se = _rmsnorm_bf16(slot_embeds.reshape(k, d), slot_norm_gamma, scale)

    slots = pl.pallas_call(
        functools.partial(_dispatch_body, ntd, k),
        grid=(bsz, ntd),
        in_specs=[
            pl.BlockSpec((1, bnd, d), lambda b, i: (b, i, 0)),
            pl.BlockSpec((k, d), lambda b, i: (0, 0)),
        ],
        out_specs=pl.BlockSpec((1, k, d), lambda b, i: (b, 0, 0)),
        out_shape=jax.ShapeDtypeStruct((bsz, k, d), jnp.float32),
        scratch_shapes=[
            pltpu.VMEM((1, k), jnp.float32),
            pltpu.VMEM((1, k), jnp.float32),
        ],
        compiler_params=pltpu.CompilerParams(
            dimension_semantics=("arbitrary", "arbitrary")),
    )(xn, se)

    y = pl.pallas_call(
        functools.partial(_ffn_body, bsz, s, d),
        grid=(e, nht),
        in_specs=[
            pl.BlockSpec((bsz, s, d), lambda i, h: (0, i, 0)),
            pl.BlockSpec((1, d, dhb), lambda i, h: (i, 0, h)),
            pl.BlockSpec((1, 1, dhb), lambda i, h: (i, 0, h)),
            pl.BlockSpec((1, dhb, d), lambda i, h: (i, h, 0)),
            pl.BlockSpec((1, 1, d), lambda i, h: (i, 0, 0)),
        ],
        out_specs=pl.BlockSpec((bsz, s, d), lambda i, h: (0, i, 0)),
        out_shape=jax.ShapeDtypeStruct((bsz, k, d), jnp.bfloat16),
        compiler_params=pltpu.CompilerParams(
            dimension_semantics=("arbitrary", "arbitrary")),
    )(slots, w1.astype(jnp.bfloat16), b1.reshape(e, 1, dh),
      w2.astype(jnp.bfloat16), b2.reshape(e, 1, d))

    out = pl.pallas_call(
        _combine_body,
        grid=(bsz, ntc),
        in_specs=[
            pl.BlockSpec((1, bnc, d), lambda b, i: (b, i, 0)),
            pl.BlockSpec((k, d), lambda b, i: (0, 0)),
            pl.BlockSpec((1, k, d), lambda b, i: (b, 0, 0)),
        ],
        out_specs=pl.BlockSpec((1, bnc, d), lambda b, i: (b, i, 0)),
        out_shape=jax.ShapeDtypeStruct((bsz, n, d), jnp.float32),
        compiler_params=pltpu.CompilerParams(
            dimension_semantics=("arbitrary", "arbitrary")),
    )(xn, se, y)
    return out


# speedup vs baseline: 1.4356x; 1.0160x over previous
import functools

import jax
import jax.numpy as jnp
from jax.experimental import pallas as pl
from jax.experimental.pallas import tpu as pltpu


def _norm_body(scale, t_ref, g_ref, o_ref):
    t = t_ref[...]
    ss = jnp.sum(t * t, axis=1, keepdims=True)
    inv = jax.lax.rsqrt(jnp.maximum(ss, 1e-24))
    o_ref[...] = (t * (inv * scale) * g_ref[...]).astype(jnp.bfloat16)


def _rmsnorm_bf16(t2, gamma, scale):
    r, d = t2.shape
    br = 512 if r % 512 == 0 else r
    return pl.pallas_call(
        functools.partial(_norm_body, scale),
        grid=(r // br,),
        in_specs=[
            pl.BlockSpec((br, d), lambda i: (i, 0)),
            pl.BlockSpec((1, d), lambda i: (0, 0)),
        ],
        out_specs=pl.BlockSpec((br, d), lambda i: (i, 0)),
        out_shape=jax.ShapeDtypeStruct((r, d), jnp.bfloat16),
    )(t2, gamma.reshape(1, d))


def _dispatch_body(xn_ref, se_ref, slots_ref):
    xn = xn_ref[0]
    se = se_ref[...]
    logits = jax.lax.dot_general(
        xn, se, (((1,), (1,)), ((), ())), preferred_element_type=jnp.float32)
    cmax = jnp.max(logits, axis=0, keepdims=True)
    p = jnp.exp(logits - cmax)
    csum = jnp.sum(p, axis=0, keepdims=True)
    p = (p * (1.0 / csum)).astype(jnp.bfloat16)
    slots_ref[0] = jax.lax.dot_general(
        p, xn, (((0,), (0,)), ((), ())),
        preferred_element_type=jnp.float32).astype(jnp.bfloat16)


def _ffn_body(bsz, s, d, slots_ref, w1_ref, b1_ref, w2_ref, b2_ref, y_ref):
    ht = pl.program_id(1)
    a = slots_ref[...].astype(jnp.bfloat16).reshape(bsz * s, d)
    h = jax.lax.dot_general(
        a, w1_ref[0], (((1,), (0,)), ((), ())),
        preferred_element_type=jnp.float32) + b1_ref[0]
    g = 0.5 * h * (1.0 + jax.lax.erf(h * 0.7071067811865476))
    part = jax.lax.dot_general(
        g.astype(jnp.bfloat16), w2_ref[0], (((1,), (0,)), ((), ())),
        preferred_element_type=jnp.float32)

    @pl.when(ht == 0)
    def _():
        y_ref[...] = (part + b2_ref[0]).astype(jnp.bfloat16).reshape(bsz, s, d)

    @pl.when(ht != 0)
    def _():
        y_ref[...] = y_ref[...] + part.astype(jnp.bfloat16).reshape(bsz, s, d)


def _combine_body(xn_ref, se_ref, y_ref, out_ref):
    xn = xn_ref[0]
    se = se_ref[...]
    y = y_ref[0]
    logits = jax.lax.dot_general(
        xn, se, (((1,), (1,)), ((), ())), preferred_element_type=jnp.float32)
    rmax = jnp.max(logits, axis=1, keepdims=True)
    c = jnp.exp(logits - rmax)
    rsum = jnp.sum(c, axis=1, keepdims=True)
    c = (c * (1.0 / rsum)).astype(jnp.bfloat16)
    out_ref[0] = jax.lax.dot_general(
        c, y, (((1,), (0,)), ((), ())), preferred_element_type=jnp.float32)


def kernel(x, norm_gamma, slot_norm_gamma, slot_embeds, w1, b1, w2, b2):
    bsz, n, d = x.shape
    e, s, _ = slot_embeds.shape
    k = e * s
    dh = w1.shape[2]
    scale = float(d) ** 0.5
    bk = 512 if k % 512 == 0 else k
    kt = k // bk
    bnc = 512 if n % 512 == 0 else n
    ntc = n // bnc
    dhb = dh // 2 if dh % 2 == 0 else dh
    nht = dh // dhb

    xn = _rmsnorm_bf16(x.reshape(bsz * n, d), norm_gamma, scale).reshape(bsz, n, d)
    se = _rmsnorm_bf16(slot_embeds.reshape(k, d), slot_norm_gamma, scale)

    slots = pl.pallas_call(
        _dispatch_body,
        grid=(bsz, kt),
        in_specs=[
            pl.BlockSpec((1, n, d), lambda b, i: (b, 0, 0)),
            pl.BlockSpec((bk, d), lambda b, i: (i, 0)),
        ],
        out_specs=pl.BlockSpec((1, bk, d), lambda b, i: (b, i, 0)),
        out_shape=jax.ShapeDtypeStruct((bsz, k, d), jnp.bfloat16),
        compiler_params=pltpu.CompilerParams(
            dimension_semantics=("parallel", "parallel")),
    )(xn, se)

    y = pl.pallas_call(
        functools.partial(_ffn_body, bsz, s, d),
        grid=(e, nht),
        in_specs=[
            pl.BlockSpec((bsz, s, d), lambda i, h: (0, i, 0)),
            pl.BlockSpec((1, d, dhb), lambda i, h: (i, 0, h)),
            pl.BlockSpec((1, 1, dhb), lambda i, h: (i, 0, h)),
            pl.BlockSpec((1, dhb, d), lambda i, h: (i, h, 0)),
            pl.BlockSpec((1, 1, d), lambda i, h: (i, 0, 0)),
        ],
        out_specs=pl.BlockSpec((bsz, s, d), lambda i, h: (0, i, 0)),
        out_shape=jax.ShapeDtypeStruct((bsz, k, d), jnp.bfloat16),
        compiler_params=pltpu.CompilerParams(
            dimension_semantics=("parallel", "arbitrary")),
    )(slots, w1.astype(jnp.bfloat16), b1.reshape(e, 1, dh),
      w2.astype(jnp.bfloat16), b2.reshape(e, 1, d))

    out = pl.pallas_call(
        _combine_body,
        grid=(bsz, ntc),
        in_specs=[
            pl.BlockSpec((1, bnc, d), lambda b, i: (b, i, 0)),
            pl.BlockSpec((k, d), lambda b, i: (0, 0)),
            pl.BlockSpec((1, k, d), lambda b, i: (b, 0, 0)),
        ],
        out_specs=pl.BlockSpec((1, bnc, d), lambda b, i: (b, i, 0)),
        out_shape=jax.ShapeDtypeStruct((bsz, n, d), jnp.float32),
        compiler_params=pltpu.CompilerParams(
            dimension_semantics=("parallel", "parallel")),
    )(xn, se, y)
    return out
